# baseline (speedup 1.0000x reference)
import functools
import math

import jax
import jax.numpy as jnp
from jax import lax
from jax.experimental import pallas as pl
from jax.experimental.pallas import tpu as pltpu
from jax.experimental.pallas import tpu_sc as plsc

N = 50000
D = 256
B = 1024

NC = 2
NS = 16
L = 16
NW = NC * NS
NJ = D // L

BLK = 80
NBLK = N // BLK
KMAX = math.ceil(NBLK / NW)
KMAX_ALL = KMAX + 2
SENT = B
BASE_BLKS = NBLK // NW
EXTRA = NBLK - BASE_BLKS * NW
SEG_PER_TILE = B // NS

RB = 2000


def _gate_body(x_ref, w_ref, b_ref, o_ref):
    x = x_ref[...]
    w = w_ref[...]
    s = jnp.dot(x, w, preferred_element_type=jnp.float32)
    o_ref[...] = jax.nn.sigmoid(s[:, 0:1] + b_ref[0, 0])


def _sc_pool_body(feats_hbm, w_hbm, seg_hbm, part_hbm, brow_hbm, bid_hbm,
                  rows_a, rows_b, w_a, w_b, idx_a, idx_b,
                  stage_v, bnd_v, ids_v, zbuf, acc_sh, sem_a, sem_b):
    cid = lax.axis_index("c")
    sid = lax.axis_index("s")
    wid = cid * NS + sid

    rows_bufs = (rows_a, rows_b)
    w_bufs = (w_a, w_b)
    idx_bufs = (idx_a, idx_b)
    sems = (sem_a, sem_b)

    def zchunk(i, carry):
        zbuf[pl.ds(i * L, L)] = jnp.zeros((L,), jnp.float32)
        return carry

    lax.fori_loop(0, (SEG_PER_TILE * D) // L, zchunk, 0)
    for j in range(2 * NJ):
        bnd_v[pl.ds(j * L, L)] = jnp.zeros((L,), jnp.float32)
    pltpu.sync_copy(zbuf, acc_sh.at[pl.ds(sid * SEG_PER_TILE * D,
                                          SEG_PER_TILE * D)])
    plsc.subcore_barrier()

    start = wid * BASE_BLKS + jnp.minimum(wid, EXTRA)
    count = BASE_BLKS + jnp.where(wid < EXTRA, 1, 0)

    def fire(k, p):
        base = (start + k) * BLK

        @pl.when(k < count)
        def _():
            pltpu.async_copy(feats_hbm.at[pl.ds(base, BLK)], rows_bufs[p],
                             sems[p])
            pltpu.async_copy(w_hbm.at[pl.ds(base, BLK)],
                             w_bufs[p].at[pl.ds(0, BLK)], sems[p])
            pltpu.async_copy(seg_hbm.at[pl.ds(base, BLK)],
                             idx_bufs[p].at[pl.ds(0, BLK)], sems[p])

    def wait_blk(k, p):
        base = (start + k) * BLK

        @pl.when(k < count)
        def _():
            pltpu.make_async_copy(feats_hbm.at[pl.ds(base, BLK)],
                                  rows_bufs[p], sems[p]).wait()
            pltpu.make_async_copy(w_hbm.at[pl.ds(base, BLK)],
                                  w_bufs[p].at[pl.ds(0, BLK)], sems[p]).wait()
            pltpu.make_async_copy(seg_hbm.at[pl.ds(base, BLK)],
                                  idx_bufs[p].at[pl.ds(0, BLK)], sems[p]).wait()

        @pl.when(k >= count)
        def _():
            for c in range((BLK + L) // L):
                w_bufs[p][pl.ds(c * L, L)] = jnp.zeros((L,), jnp.float32)
                idx_bufs[p][pl.ds(c * L, L)] = jnp.full((L,), SENT, jnp.int32)

    def flush_targets(fin, nseg, seg0v, vals):
        @pl.when(fin == seg0v)
        def _():
            for j in range(NJ):
                bnd_v[pl.ds(j * L, L)] = vals[j]

        @pl.when(jnp.logical_and(fin != seg0v, nseg >= SENT))
        def _():
            for j in range(NJ):
                bnd_v[pl.ds(D + j * L, L)] = vals[j]
            ids_v[pl.ds(0, L)] = jnp.where(
                lax.iota(jnp.int32, L) == 1, fin, 0)

        @pl.when(jnp.logical_and(fin != seg0v, nseg < SENT))
        def _():
            for j in range(NJ):
                stage_v[pl.ds(j * L, L)] = vals[j]
            pltpu.sync_copy(stage_v, acc_sh.at[pl.ds(fin * D, D)])

    def process_block(p, carry):
        rv, wv_ref, iv = rows_bufs[p], w_bufs[p], idx_bufs[p]

        def group_body(g, c2):
            prev, seg0 = c2[0], c2[1]
            acc_in = c2[2:]
            wchunk = wv_ref[pl.ds(g * L, L)]
            ichunk = iv[pl.ds(g * L, L)]
            sids = [ichunk[t] for t in range(L)]
            gates = [wchunk[t] for t in range(L)]
            prevs = [prev] + sids[:-1]
            chg = [jnp.logical_and(sids[t] != prevs[t], prevs[t] >= 0)
                   for t in range(L)]
            nchg = functools.reduce(
                jnp.add, [jnp.where(c, 1, 0) for c in chg])
            bpos = functools.reduce(
                jnp.minimum,
                [jnp.where(chg[t], t, L) for t in range(L)])
            seg0v = jnp.where(prev < 0, sids[0], seg0)

            acc = list(acc_in)
            for t in range(L):
                same = sids[t] == prevs[t]
                keep_v = lax.broadcast(
                    jnp.where(same, 1.0, 0.0).astype(jnp.float32), (L,))
                gate_v = lax.broadcast(gates[t], (L,))
                r = g * L + t
                for j in range(NJ):
                    acc[j] = acc[j] * keep_v + gate_v * rv[r, pl.ds(j * L, L)]

            @pl.when(nchg == 1)
            def _():
                pref = list(acc_in)
                for t in range(L):
                    gm = jnp.where(t < bpos, gates[t], 0.0)
                    gv = lax.broadcast(gm, (L,))
                    r = g * L + t
                    for j in range(NJ):
                        pref[j] = pref[j] + gv * rv[r, pl.ds(j * L, L)]
                fin = jnp.where(bpos == 0, prev, sids[0])
                flush_targets(fin, sids[L - 1], seg0v, tuple(pref))

            @pl.when(nchg >= 2)
            def _():
                def rb(r, cc):
                    prevr = cc[0]
                    accr = cc[1:]
                    gate = lax.broadcast(wv_ref[pl.ds(r, L)][0], (L,))
                    sid_r = iv[pl.ds(r, L)][0]
                    same = sid_r == prevr

                    @pl.when(jnp.logical_and(jnp.logical_not(same),
                                             prevr >= 0))
                    def _():
                        flush_targets(prevr, sid_r, seg0v, accr)

                    keep_v = lax.broadcast(
                        jnp.where(same, 1.0, 0.0).astype(jnp.float32), (L,))
                    new_accr = tuple(
                        accr[j] * keep_v + gate * rv[r, pl.ds(j * L, L)]
                        for j in range(NJ))
                    return (sid_r,) + new_accr

                lax.fori_loop(g * L, g * L + L, rb, (prev,) + acc_in)

            return (sids[L - 1], seg0v) + tuple(acc)

        return lax.fori_loop(0, BLK // L, group_body, carry)

    ids_v[pl.ds(0, L)] = jnp.zeros((L,), jnp.int32)
    zero_acc = tuple(jnp.zeros((L,), jnp.float32) for _ in range(NJ))
    carry = (jnp.int32(-1), jnp.int32(-1)) + zero_acc

    fire(0, 0)
    fire(1, 1)

    def pair_body(k2, carry):
        k = k2 * 2
        wait_blk(k, 0)
        carry = process_block(0, carry)
        fire(k + 2, 0)
        wait_blk(k + 1, 1)
        carry = process_block(1, carry)
        fire(k + 3, 1)
        return carry

    carry = lax.fori_loop(0, KMAX_ALL // 2, pair_body, carry)

    seg0 = carry[1]
    lanes = lax.iota(jnp.int32, L)
    ids_v[pl.ds(0, L)] = jnp.where(lanes == 0, seg0, ids_v[pl.ds(0, L)])

    pltpu.sync_copy(bnd_v, brow_hbm.at[pl.ds(wid * 2 * D, 2 * D)])
    pltpu.sync_copy(ids_v, bid_hbm.at[wid])

    plsc.subcore_barrier()
    pltpu.sync_copy(
        acc_sh.at[pl.ds(sid * SEG_PER_TILE * D, SEG_PER_TILE * D)],
        part_hbm.at[cid, pl.ds(sid * SEG_PER_TILE * D, SEG_PER_TILE * D)],
    )


def _make_sc_pool():
    return pl.kernel(
        _sc_pool_body,
        out_type=(
            jax.ShapeDtypeStruct((NC, B * D), jnp.float32),
            jax.ShapeDtypeStruct((NW * 2 * D,), jnp.float32),
            jax.ShapeDtypeStruct((NW, L), jnp.int32),
        ),
        mesh=plsc.VectorSubcoreMesh(
            core_axis_name="c", subcore_axis_name="s", num_cores=NC,
            num_subcores=NS,
        ),
        scratch_types=[
            pltpu.VMEM((BLK, D), jnp.float32),
            pltpu.VMEM((BLK, D), jnp.float32),
            pltpu.VMEM((BLK + L,), jnp.float32),
            pltpu.VMEM((BLK + L,), jnp.float32),
            pltpu.VMEM((BLK + L,), jnp.int32),
            pltpu.VMEM((BLK + L,), jnp.int32),
            pltpu.VMEM((D,), jnp.float32),
            pltpu.VMEM((2 * D,), jnp.float32),
            pltpu.VMEM((L,), jnp.int32),
            pltpu.VMEM((SEG_PER_TILE * D,), jnp.float32),
            pltpu.VMEM_SHARED((B * D,), jnp.float32),
            pltpu.SemaphoreType.DMA,
            pltpu.SemaphoreType.DMA,
        ],
    )


def _combine_body(p_ref, rows_ref, ids_ref, o_ref):
    ids2 = ids_ref[...]
    iota_b = lax.broadcasted_iota(jnp.int32, (B, 2 * NW), 0)
    onehot = (iota_b == ids2).astype(jnp.float32)
    bnd = jnp.dot(onehot, rows_ref[...],
                  preferred_element_type=jnp.float32)
    o_ref[...] = p_ref[0] + p_ref[1] + bnd


def kernel(feats, segment_ids, W, b):
    seg_i32 = segment_ids.astype(jnp.int32)

    w2d = pl.pallas_call(
        _gate_body,
        grid=(N // RB,),
        in_specs=[
            pl.BlockSpec((RB, D), lambda i: (i, 0)),
            pl.BlockSpec((D, 128), lambda i: (0, 0)),
            pl.BlockSpec((1, 1), lambda i: (0, 0)),
        ],
        out_specs=pl.BlockSpec((RB, 1), lambda i: (i, 0)),
        out_shape=jax.ShapeDtypeStruct((N, 1), jnp.float32),
    )(feats, jnp.broadcast_to(W, (D, 128)), b.reshape(1, 1))
    w_flat = w2d.reshape(N)

    partials_f, brows_f, bids = _make_sc_pool()(feats, w_flat, seg_i32)
    partials = partials_f.reshape(NC, B, D)
    brows = brows_f.reshape(NW * 2, D)
    ids_flat = bids[:, :2].reshape(1, 2 * NW)

    out = pl.pallas_call(
        _combine_body,
        out_shape=jax.ShapeDtypeStruct((B, D), jnp.float32),
    )(partials, brows, ids_flat)
    return out


# speedup vs baseline: 1.6031x; 1.6031x over previous
import functools
import math

import jax
import jax.numpy as jnp
from jax import lax
from jax.experimental import pallas as pl
from jax.experimental.pallas import tpu as pltpu
from jax.experimental.pallas import tpu_sc as plsc

N = 50000
D = 256
B = 1024

NC = 2
NS = 16
L = 16
NW = NC * NS
NJ = D // L

BLK = 80
NBLK = N // BLK
KMAX = math.ceil(NBLK / NW)
KMAX_ALL = KMAX + 2
SENT = B
BASE_BLKS = NBLK // NW
EXTRA = NBLK - BASE_BLKS * NW
SEG_PER_TILE = B // NS

RB = 2000


def _gate_body(x_ref, w_ref, b_ref, o_ref):
    x = x_ref[...]
    w = w_ref[...]
    s = jnp.sum(x * w, axis=1) + b_ref[0, 0]
    o_ref[...] = jax.nn.sigmoid(s)[None, None, :]


def _sc_pool_body(feats_hbm, w_hbm, seg_hbm, part_hbm, brow_hbm, bid_hbm,
                  rows_a, rows_b, w_a, w_b, idx_a, idx_b,
                  stage_v, bnd_v, ids_v, zbuf, acc_sh, sem_a, sem_b):
    cid = lax.axis_index("c")
    sid = lax.axis_index("s")
    wid = cid * NS + sid

    rows_bufs = (rows_a, rows_b)
    w_bufs = (w_a, w_b)
    idx_bufs = (idx_a, idx_b)
    sems = (sem_a, sem_b)

    def zchunk(i, carry):
        zbuf[pl.ds(i * L, L)] = jnp.zeros((L,), jnp.float32)
        return carry

    lax.fori_loop(0, (SEG_PER_TILE * D) // L, zchunk, 0)
    for j in range(2 * NJ):
        bnd_v[pl.ds(j * L, L)] = jnp.zeros((L,), jnp.float32)
    pltpu.sync_copy(zbuf, acc_sh.at[pl.ds(sid * SEG_PER_TILE * D,
                                          SEG_PER_TILE * D)])
    plsc.subcore_barrier()

    start = wid * BASE_BLKS + jnp.minimum(wid, EXTRA)
    count = BASE_BLKS + jnp.where(wid < EXTRA, 1, 0)

    def fire(k, p):
        base = (start + k) * BLK

        @pl.when(k < count)
        def _():
            pltpu.async_copy(feats_hbm.at[pl.ds(base, BLK)], rows_bufs[p],
                             sems[p])
            pltpu.async_copy(w_hbm.at[pl.ds(base, BLK)],
                             w_bufs[p].at[pl.ds(0, BLK)], sems[p])
            pltpu.async_copy(seg_hbm.at[pl.ds(base, BLK)],
                             idx_bufs[p].at[pl.ds(0, BLK)], sems[p])

    def wait_blk(k, p):
        base = (start + k) * BLK

        @pl.when(k < count)
        def _():
            pltpu.make_async_copy(feats_hbm.at[pl.ds(base, BLK)],
                                  rows_bufs[p], sems[p]).wait()
            pltpu.make_async_copy(w_hbm.at[pl.ds(base, BLK)],
                                  w_bufs[p].at[pl.ds(0, BLK)], sems[p]).wait()
            pltpu.make_async_copy(seg_hbm.at[pl.ds(base, BLK)],
                                  idx_bufs[p].at[pl.ds(0, BLK)], sems[p]).wait()

        @pl.when(k >= count)
        def _():
            for c in range((BLK + L) // L):
                w_bufs[p][pl.ds(c * L, L)] = jnp.zeros((L,), jnp.float32)
                idx_bufs[p][pl.ds(c * L, L)] = jnp.full((L,), SENT, jnp.int32)

    def process_block(p, carry):
        rv, wv_ref, iv = rows_bufs[p], w_bufs[p], idx_bufs[p]

        def row_body(r, c2):
            prev, seg0 = c2[0], c2[1]
            acc = c2[2:]
            gate = lax.broadcast(wv_ref[pl.ds(r, L)][0], (L,))
            sid_r = iv[pl.ds(r, L)][0]
            rows = [rv[r, pl.ds(j * L, L)] for j in range(NJ)]

            same = sid_r == prev
            do_flush = jnp.logical_and(jnp.logical_not(same), prev >= 0)

            @pl.when(do_flush)
            def _():
                @pl.when(prev == seg0)
                def _():
                    for j in range(NJ):
                        bnd_v[pl.ds(j * L, L)] = acc[j]

                @pl.when(jnp.logical_and(prev != seg0, sid_r >= SENT))
                def _():
                    for j in range(NJ):
                        bnd_v[pl.ds(D + j * L, L)] = acc[j]
                    ids_v[pl.ds(0, L)] = jnp.where(
                        lax.iota(jnp.int32, L) == 1, prev, 0)

                @pl.when(jnp.logical_and(prev != seg0, sid_r < SENT))
                def _():
                    for j in range(NJ):
                        stage_v[pl.ds(j * L, L)] = acc[j]
                    pltpu.sync_copy(stage_v, acc_sh.at[pl.ds(prev * D, D)])

            keep_v = lax.broadcast(
                jnp.where(same, 1.0, 0.0).astype(jnp.float32), (L,))
            new_acc = tuple(acc[j] * keep_v + gate * rows[j]
                            for j in range(NJ))
            return (sid_r, jnp.where(prev < 0, sid_r, seg0)) + new_acc

        return lax.fori_loop(0, BLK, row_body, carry)

    ids_v[pl.ds(0, L)] = jnp.zeros((L,), jnp.int32)
    zero_acc = tuple(jnp.zeros((L,), jnp.float32) for _ in range(NJ))
    carry = (jnp.int32(-1), jnp.int32(-1)) + zero_acc

    fire(0, 0)
    fire(1, 1)

    def pair_body(k2, carry):
        k = k2 * 2
        wait_blk(k, 0)
        carry = process_block(0, carry)
        fire(k + 2, 0)
        wait_blk(k + 1, 1)
        carry = process_block(1, carry)
        fire(k + 3, 1)
        return carry

    carry = lax.fori_loop(0, KMAX_ALL // 2, pair_body, carry)

    seg0 = carry[1]
    lanes = lax.iota(jnp.int32, L)
    ids_v[pl.ds(0, L)] = jnp.where(lanes == 0, seg0, ids_v[pl.ds(0, L)])

    pltpu.sync_copy(bnd_v, brow_hbm.at[pl.ds(wid * 2 * D, 2 * D)])
    pltpu.sync_copy(ids_v, bid_hbm.at[wid])

    plsc.subcore_barrier()
    pltpu.sync_copy(
        acc_sh.at[pl.ds(sid * SEG_PER_TILE * D, SEG_PER_TILE * D)],
        part_hbm.at[cid, pl.ds(sid * SEG_PER_TILE * D, SEG_PER_TILE * D)],
    )


def _make_sc_pool():
    return pl.kernel(
        _sc_pool_body,
        out_type=(
            jax.ShapeDtypeStruct((NC, B * D), jnp.float32),
            jax.ShapeDtypeStruct((NW * 2 * D,), jnp.float32),
            jax.ShapeDtypeStruct((NW, L), jnp.int32),
        ),
        mesh=plsc.VectorSubcoreMesh(
            core_axis_name="c", subcore_axis_name="s", num_cores=NC,
            num_subcores=NS,
        ),
        scratch_types=[
            pltpu.VMEM((BLK, D), jnp.float32),
            pltpu.VMEM((BLK, D), jnp.float32),
            pltpu.VMEM((BLK + L,), jnp.float32),
            pltpu.VMEM((BLK + L,), jnp.float32),
            pltpu.VMEM((BLK + L,), jnp.int32),
            pltpu.VMEM((BLK + L,), jnp.int32),
            pltpu.VMEM((D,), jnp.float32),
            pltpu.VMEM((2 * D,), jnp.float32),
            pltpu.VMEM((L,), jnp.int32),
            pltpu.VMEM((SEG_PER_TILE * D,), jnp.float32),
            pltpu.VMEM_SHARED((B * D,), jnp.float32),
            pltpu.SemaphoreType.DMA,
            pltpu.SemaphoreType.DMA,
        ],
    )


def _combine_body(p_ref, rows_ref, ids_ref, o_ref):
    ids2 = ids_ref[...]
    iota_b = lax.broadcasted_iota(jnp.int32, (B, 2 * NW), 0)
    onehot = (iota_b == ids2).astype(jnp.float32)
    bnd = jnp.dot(onehot, rows_ref[...],
                  preferred_element_type=jnp.float32)
    o_ref[...] = p_ref[0] + p_ref[1] + bnd


def kernel(feats, segment_ids, W, b):
    seg_i32 = segment_ids.astype(jnp.int32)

    w2d = pl.pallas_call(
        _gate_body,
        grid=(N // RB,),
        in_specs=[
            pl.BlockSpec((RB, D), lambda i: (i, 0)),
            pl.BlockSpec((1, D), lambda i: (0, 0)),
            pl.BlockSpec((1, 1), lambda i: (0, 0)),
        ],
        out_specs=pl.BlockSpec((1, 1, RB), lambda i: (i, 0, 0)),
        out_shape=jax.ShapeDtypeStruct((N // RB, 1, RB), jnp.float32),
    )(feats, W.reshape(1, D), b.reshape(1, 1))
    w_flat = w2d.reshape(N)

    partials_f, brows_f, bids = _make_sc_pool()(feats, w_flat, seg_i32)
    partials = partials_f.reshape(NC, B, D)
    brows = brows_f.reshape(NW * 2, D)
    ids_flat = bids[:, :2].reshape(1, 2 * NW)

    out = pl.pallas_call(
        _combine_body,
        out_shape=jax.ShapeDtypeStruct((B, D), jnp.float32),
    )(partials, brows, ids_flat)
    return out
